# Initial kernel scaffold; baseline (speedup 1.0000x reference)
import jax
import jax.numpy as jnp
from jax.experimental import pallas as pl


def kernel(pred_objectness_logits, pred_anchor_deltas, anchors):
    raise NotImplementedError("write your pallas kernel here")


# speedup vs baseline: 10.7981x; 10.7981x over previous
import math

import jax
import jax.numpy as jnp
from jax import lax
from jax.experimental import pallas as pl
from jax.experimental.pallas import tpu as pltpu

IMG_H = 1024.0
IMG_W = 1024.0
PRE_NMS_TOPK = 2000
POST_NMS_TOPK = 1000
NMS_THRESH = 0.7
SCALE_CLAMP = math.log(1000.0 / 16.0)

P = 2048
T = 128
NB = P // T
LN = 128


def _run_bitonic(data_ref, afirst_fn):
    C, B, Q, L = data_ref.shape
    M = Q * L
    p = M.bit_length() - 1
    nstages = p * (p + 1) // 2
    gi = (lax.broadcasted_iota(jnp.int32, (1, 1, Q, L), 2) * L
          + lax.broadcasted_iota(jnp.int32, (1, 1, Q, L), 3))

    def stage(_, ks):
        k, s = ks
        x = data_ref[...]
        d2 = s // L
        d3 = s % L
        fwd = pltpu.roll(pltpu.roll(x, (Q - d2) % Q, 2), (L - d3) % L, 3)
        bwd = pltpu.roll(pltpu.roll(x, d2, 2), d3, 3)
        own_a = (gi & s) == 0
        oth = jnp.where(own_a, fwd, bwd)
        pair_a = jnp.where(own_a, x, oth)
        pair_b = jnp.where(own_a, oth, x)
        afirst = afirst_fn(pair_a, pair_b)
        dirm = (gi & k) == 0
        stay = jnp.where(afirst, 1, 0) == jnp.where(dirm, 1, 0)
        data_ref[...] = jnp.where(stay, x, oth)
        s2 = s // 2
        done = s2 < 1
        k2 = jnp.where(done, k * 2, k)
        s3 = jnp.where(done, k, s2)
        return (k2, s3)

    lax.fori_loop(0, nstages, stage, (jnp.int32(2), jnp.int32(1)))


def _afirst_score(a, b):
    va, vb = a[0:1], b[0:1]
    ia, ib = a[1:2], b[1:2]
    return (va > vb) | ((va == vb) & (ia < ib))


def _afirst_key_asc(a, b):
    return a[0:1] < b[0:1]


def _iou(b1, b2, a1, a2):
    xx1 = jnp.maximum(b1[0][:, :, None], b2[0][:, None, :])
    yy1 = jnp.maximum(b1[1][:, :, None], b2[1][:, None, :])
    xx2 = jnp.minimum(b1[2][:, :, None], b2[2][:, None, :])
    yy2 = jnp.minimum(b1[3][:, :, None], b2[3][:, None, :])
    inter = jnp.maximum(xx2 - xx1, 0.0) * jnp.maximum(yy2 - yy1, 0.0)
    return inter / jnp.maximum(a1[:, :, None] + a2[:, None, :] - inter, 1e-6)


def _read_blk(cref, keep_ref, blk):
    coords = tuple(cref[c, :, pl.ds(blk, 1), :][:, 0, :] for c in range(4))
    area = cref[4, :, pl.ds(blk, 1), :][:, 0, :]
    keep = keep_ref[:, pl.ds(blk, 1), :][:, 0, :]
    return coords, area, keep


def _body(logits_ref, deltas_ref, anchors_ref, boxes_ref, scores_ref,
          data_ref, d2_ref, cref, keep_ref, sup_ref):
    B, N = logits_ref.shape
    _, _, Q, L = data_ref.shape
    M = Q * L

    ax1 = anchors_ref[0:1, :]
    ay1 = anchors_ref[1:2, :]
    ax2 = anchors_ref[2:3, :]
    ay2 = anchors_ref[3:4, :]
    w = ax2 - ax1
    h = ay2 - ay1
    cx = ax1 + 0.5 * w
    cy = ay1 + 0.5 * h
    dx = deltas_ref[:, 0, :]
    dy = deltas_ref[:, 1, :]
    dw = jnp.minimum(deltas_ref[:, 2, :], SCALE_CLAMP)
    dh = jnp.minimum(deltas_ref[:, 3, :], SCALE_CLAMP)
    pcx = dx * w + cx
    pcy = dy * h + cy
    pw = jnp.exp(dw) * w
    ph = jnp.exp(dh) * h
    x1 = jnp.clip(pcx - 0.5 * pw, 0.0, IMG_W)
    y1 = jnp.clip(pcy - 0.5 * ph, 0.0, IMG_H)
    x2 = jnp.clip(pcx + 0.5 * pw, 0.0, IMG_W)
    y2 = jnp.clip(pcy + 0.5 * ph, 0.0, IMG_H)

    pad = M - N
    ninf = jnp.full((B, pad), -jnp.inf, jnp.float32)
    zero = jnp.zeros((B, pad), jnp.float32)
    idxf = lax.broadcasted_iota(jnp.int32, (B, M), 1).astype(jnp.float32)
    data_ref[...] = jnp.stack([
        jnp.concatenate([logits_ref[...], ninf], axis=1),
        idxf,
        jnp.concatenate([x1, zero], axis=1),
        jnp.concatenate([y1, zero], axis=1),
        jnp.concatenate([x2, zero], axis=1),
        jnp.concatenate([y2, zero], axis=1),
    ], axis=0).reshape(6, B, Q, L)
    _run_bitonic(data_ref, _afirst_score)

    top = data_ref[:, :, :NB, :]
    scores = top[1 - 1]
    tx1, ty1, tx2, ty2 = top[2], top[3], top[4], top[5]

    bw = tx2 - tx1
    bh = ty2 - ty1
    area = bw * bh
    gpos = (lax.broadcasted_iota(jnp.int32, (1, NB, LN), 1) * LN
            + lax.broadcasted_iota(jnp.int32, (1, NB, LN), 2))
    keep0 = (bw > 0.0) & (bh > 0.0) & (gpos < PRE_NMS_TOPK)
    cref[...] = jnp.stack([tx1, ty1, tx2, ty2, area], axis=0)
    keep_ref[...] = jnp.where(keep0, 1.0, 0.0)

    iblk = lax.broadcasted_iota(jnp.int32, (1, T, T), 1)
    jblk = lax.broadcasted_iota(jnp.int32, (1, T, T), 2)
    jgt = jblk > iblk
    lane = lax.broadcasted_iota(jnp.int32, (1, T), 1)

    def blk_body(blk, carry):
        b1, a1, k1 = _read_blk(cref, keep_ref, blk)
        iou = _iou(b1, b1, a1, a1)
        sup_ref[...] = jnp.where((iou > NMS_THRESH) & jgt, 1.0, 0.0)

        def inner(i, kvf):
            rowf = sup_ref[:, pl.ds(i, 1), :][:, 0, :]
            onei = lane == i
            kif = jnp.sum(jnp.where(onei, kvf, 0.0), axis=1, keepdims=True)
            return kvf * jnp.where((rowf * kif) > 0.5, 0.0, 1.0)

        kb = lax.fori_loop(0, T, inner, k1)
        keep_ref[:, pl.ds(blk, 1), :] = kb.reshape(B, 1, T)
        kbf3 = kb[:, :, None]

        def cross(lb, c2):
            b2, a2, k2 = _read_blk(cref, keep_ref, lb)
            iou2 = _iou(b1, b2, a1, a2)
            supx = jnp.max(jnp.where(iou2 > NMS_THRESH, kbf3, 0.0), axis=1)
            newk = k2 * jnp.where(supx > 0.5, 0.0, 1.0)
            keep_ref[:, pl.ds(lb, 1), :] = newk.reshape(B, 1, T)
            return c2

        lax.fori_loop(blk + 1, NB, cross, 0)
        return carry

    lax.fori_loop(0, NB, blk_body, 0)

    keep = keep_ref[...] > 0.5
    posf = gpos.astype(jnp.float32)
    ckey = jnp.where(keep, posf, float(2 * P) + posf)
    sc2 = jnp.where(keep, scores, -1e9)
    d2_ref[...] = jnp.stack([ckey, sc2, tx1, ty1, tx2, ty2], axis=0)
    _run_bitonic(d2_ref, _afirst_key_asc)

    out = d2_ref[...].reshape(6, B, P)
    scores_ref[...] = out[1][:, :POST_NMS_TOPK]
    for c in range(4):
        boxes_ref[:, c, :] = out[2 + c][:, :POST_NMS_TOPK]


def kernel(pred_objectness_logits, pred_anchor_deltas, anchors):
    B, N = pred_objectness_logits.shape
    M = 1
    while M < N:
        M *= 2
    lg = pred_objectness_logits.astype(jnp.float32)
    dT = pred_anchor_deltas.astype(jnp.float32).transpose(0, 2, 1)
    aT = anchors.astype(jnp.float32).T
    boxesT, scores = pl.pallas_call(
        _body,
        out_shape=(
            jax.ShapeDtypeStruct((B, 4, POST_NMS_TOPK), jnp.float32),
            jax.ShapeDtypeStruct((B, POST_NMS_TOPK), jnp.float32),
        ),
        scratch_shapes=[
            pltpu.VMEM((6, B, M // LN, LN), jnp.float32),
            pltpu.VMEM((6, B, NB, LN), jnp.float32),
            pltpu.VMEM((5, B, NB, LN), jnp.float32),
            pltpu.VMEM((B, NB, LN), jnp.float32),
            pltpu.VMEM((B, T, T), jnp.float32),
        ],
    )(lg, dT, aT)
    return boxesT.transpose(0, 2, 1), scores


# speedup vs baseline: 14.6261x; 1.3545x over previous
import functools
import math

import jax
import jax.numpy as jnp
from jax import lax
from jax.experimental import pallas as pl
from jax.experimental.pallas import tpu as pltpu
from jax.experimental.pallas import tpu_sc as plsc

IMG_H = 1024.0
IMG_W = 1024.0
PRE_NMS_TOPK = 2000
POST_NMS_TOPK = 1000
NMS_THRESH = 0.7
SCALE_CLAMP = math.log(1000.0 / 16.0)

P = 2048
T = 128
NB = P // T
LN = 128


def _stage_tail(data_ref, x, fwd, bwd, gi, s, dirm, afirst_fn):
    own_a = (gi & s) == 0
    oth = jnp.where(own_a, fwd, bwd)
    pair_a = jnp.where(own_a, x, oth)
    pair_b = jnp.where(own_a, oth, x)
    afirst = afirst_fn(pair_a, pair_b)
    stay = jnp.where(afirst, 1, 0) == jnp.where(dirm, 1, 0)
    data_ref[...] = jnp.where(stay, x, oth)


def _run_bitonic(data_ref, afirst_fn):
    C, B, Q, L = data_ref.shape
    M = Q * L
    gi = (lax.broadcasted_iota(jnp.int32, (1, 1, Q, L), 2) * L
          + lax.broadcasted_iota(jnp.int32, (1, 1, Q, L), 3))

    k = 2
    while k <= M:
        s0 = k // 2
        dirm = (gi & k) == 0

        def coarse(_, s, dirm=dirm):
            x = data_ref[...]
            d2 = s // L
            fwd = pltpu.roll(x, (Q - d2) % Q, 2)
            bwd = pltpu.roll(x, d2, 2)
            _stage_tail(data_ref, x, fwd, bwd, gi, s, dirm, afirst_fn)
            return s // 2

        def fine(_, s, dirm=dirm):
            x = data_ref[...]
            fwd = pltpu.roll(x, (L - s) % L, 3)
            bwd = pltpu.roll(x, s, 3)
            _stage_tail(data_ref, x, fwd, bwd, gi, s, dirm, afirst_fn)
            return s // 2

        p0 = s0.bit_length() - 1
        n_coarse = max(0, p0 - 6)
        n_fine = min(p0 + 1, 7)
        s_cur = jnp.int32(s0)
        if n_coarse:
            s_cur = lax.fori_loop(0, n_coarse, coarse, s_cur)
        lax.fori_loop(0, n_fine, fine, s_cur)
        k *= 2


def _afirst_score(a, b):
    va, vb = a[0:1], b[0:1]
    ia, ib = a[1:2], b[1:2]
    return (va > vb) | ((va == vb) & (ia < ib))


def _afirst_key_asc(a, b):
    return a[0:1] < b[0:1]


def _topk_body(logits_ref, scores_ref, idx_ref, data_ref):
    B, N = logits_ref.shape
    _, _, Q, L = data_ref.shape
    M = Q * L
    pad = M - N
    ninf = jnp.full((B, pad), -jnp.inf, jnp.float32)
    idxf = lax.broadcasted_iota(jnp.int32, (B, M), 1).astype(jnp.float32)
    data_ref[...] = jnp.stack([
        jnp.concatenate([logits_ref[...], ninf], axis=1),
        idxf,
    ], axis=0).reshape(2, B, Q, L)
    _run_bitonic(data_ref, _afirst_score)
    top = data_ref[:, :, :NB, :].reshape(2, B, P)
    scores_ref[...] = top[0]
    bofs = lax.broadcasted_iota(jnp.int32, (B, P), 0) * N
    idx_ref[...] = top[1].astype(jnp.int32) + bofs


def _iou(b1, b2, a1, a2):
    xx1 = jnp.maximum(b1[0][:, :, None], b2[0][:, None, :])
    yy1 = jnp.maximum(b1[1][:, :, None], b2[1][:, None, :])
    xx2 = jnp.minimum(b1[2][:, :, None], b2[2][:, None, :])
    yy2 = jnp.minimum(b1[3][:, :, None], b2[3][:, None, :])
    inter = jnp.maximum(xx2 - xx1, 0.0) * jnp.maximum(yy2 - yy1, 0.0)
    return inter / jnp.maximum(a1[:, :, None] + a2[:, None, :] - inter, 1e-6)


def _read_blk(cref, keep_ref, blk):
    coords = tuple(cref[c, :, pl.ds(blk, 1), :][:, 0, :] for c in range(4))
    area = cref[4, :, pl.ds(blk, 1), :][:, 0, :]
    keep = keep_ref[:, pl.ds(blk, 1), :][:, 0, :]
    return coords, area, keep


def _nms_body(scores_in_ref, g_ref, boxes_ref, scores_ref,
              d2_ref, cref, keep_ref, sup_ref):
    B, _ = scores_in_ref.shape
    scores = scores_in_ref[...].reshape(B, NB, LN)

    ax1 = g_ref[:, 0, :]
    ay1 = g_ref[:, 1, :]
    ax2 = g_ref[:, 2, :]
    ay2 = g_ref[:, 3, :]
    dx = g_ref[:, 4, :]
    dy = g_ref[:, 5, :]
    dw = jnp.minimum(g_ref[:, 6, :], SCALE_CLAMP)
    dh = jnp.minimum(g_ref[:, 7, :], SCALE_CLAMP)
    w = ax2 - ax1
    h = ay2 - ay1
    cx = ax1 + 0.5 * w
    cy = ay1 + 0.5 * h
    pcx = dx * w + cx
    pcy = dy * h + cy
    pw = jnp.exp(dw) * w
    ph = jnp.exp(dh) * h
    tx1 = jnp.clip(pcx - 0.5 * pw, 0.0, IMG_W).reshape(B, NB, LN)
    ty1 = jnp.clip(pcy - 0.5 * ph, 0.0, IMG_H).reshape(B, NB, LN)
    tx2 = jnp.clip(pcx + 0.5 * pw, 0.0, IMG_W).reshape(B, NB, LN)
    ty2 = jnp.clip(pcy + 0.5 * ph, 0.0, IMG_H).reshape(B, NB, LN)

    bw = tx2 - tx1
    bh = ty2 - ty1
    area = bw * bh
    gpos = (lax.broadcasted_iota(jnp.int32, (1, NB, LN), 1) * LN
            + lax.broadcasted_iota(jnp.int32, (1, NB, LN), 2))
    keep0 = (bw > 0.0) & (bh > 0.0) & (gpos < PRE_NMS_TOPK)
    cref[...] = jnp.stack([tx1, ty1, tx2, ty2, area], axis=0)
    keep_ref[...] = jnp.where(keep0, 1.0, 0.0)

    iblk = lax.broadcasted_iota(jnp.int32, (1, T, T), 1)
    jblk = lax.broadcasted_iota(jnp.int32, (1, T, T), 2)
    jgt = jblk > iblk
    lane = lax.broadcasted_iota(jnp.int32, (1, T), 1)

    def blk_body(blk, carry):
        b1, a1, k1 = _read_blk(cref, keep_ref, blk)
        iou = _iou(b1, b1, a1, a1)
        sup_ref[...] = jnp.where((iou > NMS_THRESH) & jgt, 1.0, 0.0)

        def inner(i, kvf):
            rowf = sup_ref[:, pl.ds(i, 1), :][:, 0, :]
            onei = lane == i
            kif = jnp.sum(jnp.where(onei, kvf, 0.0), axis=1, keepdims=True)
            return kvf * jnp.where((rowf * kif) > 0.5, 0.0, 1.0)

        kb = lax.fori_loop(0, T, inner, k1)
        keep_ref[:, pl.ds(blk, 1), :] = kb.reshape(B, 1, T)
        kbf3 = kb[:, :, None]

        def cross(lb, c2):
            b2, a2, k2 = _read_blk(cref, keep_ref, lb)
            iou2 = _iou(b1, b2, a1, a2)
            supx = jnp.max(jnp.where(iou2 > NMS_THRESH, kbf3, 0.0), axis=1)
            newk = k2 * jnp.where(supx > 0.5, 0.0, 1.0)
            keep_ref[:, pl.ds(lb, 1), :] = newk.reshape(B, 1, T)
            return c2

        lax.fori_loop(blk + 1, NB, cross, 0)
        return carry

    lax.fori_loop(0, NB, blk_body, 0)

    keep = keep_ref[...] > 0.5
    posf = gpos.astype(jnp.float32)
    ckey = jnp.where(keep, posf, float(2 * P) + posf)
    sc2 = jnp.where(keep, scores, -1e9)
    d2_ref[...] = jnp.stack([ckey, sc2, tx1, ty1, tx2, ty2], axis=0)
    _run_bitonic(d2_ref, _afirst_key_asc)

    out = d2_ref[...].reshape(6, B, P)
    scores_ref[...] = out[1][:, :POST_NMS_TOPK]
    for c in range(4):
        boxes_ref[:, c, :] = out[2 + c][:, :POST_NMS_TOPK]


def _make_sc_gather(rows, width):
    info = plsc.get_sparse_core_info()
    nw = info.num_cores * info.num_subcores
    rpw = rows // nw
    chunk = min(rpw, 128)
    mesh = plsc.VectorSubcoreMesh(core_axis_name="c", subcore_axis_name="s")

    @functools.partial(
        pl.kernel, mesh=mesh,
        out_type=jax.ShapeDtypeStruct((rows, width), jnp.float32),
        scratch_types=[
            pltpu.VMEM((chunk,), jnp.int32),
            pltpu.VMEM((chunk, width), jnp.float32),
            pltpu.SemaphoreType.DMA,
        ],
    )
    def gather_k(table_hbm, idx_hbm, out_hbm, idx_v, rows_v, sem):
        wid = lax.axis_index("s") * info.num_cores + lax.axis_index("c")
        for j in range(rpw // chunk):
            base = wid * rpw + j * chunk
            pltpu.sync_copy(idx_hbm.at[pl.ds(base, chunk)], idx_v)
            pltpu.async_copy(table_hbm.at[idx_v], rows_v, sem).wait()
            pltpu.sync_copy(rows_v, out_hbm.at[pl.ds(base, chunk)])

    return gather_k


def kernel(pred_objectness_logits, pred_anchor_deltas, anchors):
    B, N = pred_objectness_logits.shape
    M = 1
    while M < N:
        M *= 2
    lg = pred_objectness_logits.astype(jnp.float32)
    deltas = pred_anchor_deltas.astype(jnp.float32)
    anc = anchors.astype(jnp.float32)

    scores_p, idx_flat = pl.pallas_call(
        _topk_body,
        out_shape=(
            jax.ShapeDtypeStruct((B, P), jnp.float32),
            jax.ShapeDtypeStruct((B, P), jnp.int32),
        ),
        scratch_shapes=[pltpu.VMEM((2, B, M // LN, LN), jnp.float32)],
    )(lg)

    table = jnp.concatenate(
        [jnp.broadcast_to(anc[None, :, :], (B, N, 4)), deltas,
         jnp.zeros((B, N, 120), jnp.float32)], axis=2
    ).reshape(B * N, 128)
    gathered = _make_sc_gather(B * P, 128)(table, idx_flat.reshape(B * P))
    g = gathered.reshape(B, P, 128)[:, :, :8].transpose(0, 2, 1)

    boxesT, scores = pl.pallas_call(
        _nms_body,
        out_shape=(
            jax.ShapeDtypeStruct((B, 4, POST_NMS_TOPK), jnp.float32),
            jax.ShapeDtypeStruct((B, POST_NMS_TOPK), jnp.float32),
        ),
        scratch_shapes=[
            pltpu.VMEM((6, B, NB, LN), jnp.float32),
            pltpu.VMEM((5, B, NB, LN), jnp.float32),
            pltpu.VMEM((B, NB, LN), jnp.float32),
            pltpu.VMEM((B, T, T), jnp.float32),
        ],
    )(scores_p, g)
    return boxesT.transpose(0, 2, 1), scores
